# baseline (device time: 47944 ns/iter reference)
import jax
import jax.numpy as jnp
from jax import lax
from jax.experimental import pallas as pl
from jax.experimental.pallas import tpu as pltpu

N_DEV = 32
T = 1024
D = 256
E = 128
H = 512
E_LOCAL = E // N_DEV
TPB = T // N_DEV


def kernel(x, router_W, route_idx, expert_W, shared_W):
    def body(x_ref, rW_ref, idx_ref, eW_ref, sW_ref, out_ref,
             send_buf, acc_ref, send_sems, recv_sems):
        my = lax.axis_index("i")

        xv = x_ref[...]
        scores = jnp.dot(xv, rW_ref[...], preferred_element_type=jnp.float32)
        smax = jnp.max(scores, axis=-1, keepdims=True)
        p = jnp.exp(scores - smax)
        p = p / jnp.sum(p, axis=-1, keepdims=True)
        route = idx_ref[...]
        sel = lax.broadcasted_iota(jnp.int32, (T, E), 1) == route
        w = jnp.sum(jnp.where(sel, p, 0.0), axis=-1, keepdims=True)

        partial = jnp.zeros((T, H), jnp.float32)
        for el in range(E_LOCAL):
            ge = my * E_LOCAL + el
            mask = jnp.where(route == ge, w, 0.0)
            partial = partial + jnp.dot(
                xv * mask, eW_ref[el], preferred_element_type=jnp.float32)
        send_buf[...] = partial

        acc_ref[pl.ds(my, 1)] = send_buf[pl.ds(my * TPB, TPB), :][None]

        for j in range(N_DEV):
            @pl.when(j != my)
            def _(j=j):
                pltpu.make_async_remote_copy(
                    src_ref=send_buf.at[pl.ds(j * TPB, TPB)],
                    dst_ref=acc_ref.at[my],
                    send_sem=send_sems.at[j],
                    recv_sem=recv_sems.at[my],
                    device_id=(j,),
                    device_id_type=pltpu.DeviceIdType.MESH,
                ).start()

        shared = jnp.dot(x_ref[pl.ds(my * TPB, TPB), :], sW_ref[...],
                         preferred_element_type=jnp.float32)

        for s in range(N_DEV):
            @pl.when(s != my)
            def _(s=s):
                pltpu.make_async_remote_copy(
                    src_ref=send_buf.at[pl.ds(0, TPB)],
                    dst_ref=acc_ref.at[s],
                    send_sem=send_sems.at[s],
                    recv_sem=recv_sems.at[s],
                    device_id=(s,),
                    device_id_type=pltpu.DeviceIdType.MESH,
                ).wait_recv()

        out_ref[...] = shared + jnp.sum(acc_ref[...], axis=0)

        for j in range(N_DEV):
            @pl.when(j != my)
            def _(j=j):
                pltpu.make_async_remote_copy(
                    src_ref=send_buf.at[pl.ds(j * TPB, TPB)],
                    dst_ref=acc_ref.at[my],
                    send_sem=send_sems.at[j],
                    recv_sem=recv_sems.at[my],
                    device_id=(j,),
                    device_id_type=pltpu.DeviceIdType.MESH,
                ).wait_send()

    return pl.pallas_call(
        body,
        out_shape=jax.ShapeDtypeStruct((TPB, H), jnp.float32),
        in_specs=[pl.BlockSpec(memory_space=pltpu.VMEM)] * 5,
        out_specs=pl.BlockSpec(memory_space=pltpu.VMEM),
        scratch_shapes=[
            pltpu.VMEM((T, H), jnp.float32),
            pltpu.VMEM((N_DEV, TPB, H), jnp.float32),
            pltpu.SemaphoreType.DMA((N_DEV,)),
            pltpu.SemaphoreType.DMA((N_DEV,)),
        ],
    )(x, router_W, route_idx, expert_W, shared_W)
